# baseline (speedup 1.0000x reference)
import functools

import jax
import jax.numpy as jnp
from jax.experimental import pallas as pl
from jax.experimental.pallas import tpu as pltpu

_BN = 1000


def _fused_linears_kernel(x_ref, wc_ref, bc_ref, wb_ref, bb_ref,
                          scores_ref, deltas_ref):
    x = x_ref[...].astype(jnp.bfloat16)
    scores_ref[...] = (
        jnp.dot(x, wc_ref[...].astype(jnp.bfloat16),
                preferred_element_type=jnp.float32)
        + bc_ref[...]
    )
    deltas_ref[...] = (
        jnp.dot(x, wb_ref[...].astype(jnp.bfloat16),
                preferred_element_type=jnp.float32)
        + bb_ref[...]
    )


@jax.jit
def kernel(x, W_cls, b_cls, W_box, b_box):
    if x.ndim > 2:
        x = x.reshape((x.shape[0], -1))
    n, d = x.shape
    kc = W_cls.shape[1]
    kb = W_box.shape[1]
    grid = (n // _BN,)
    scores, deltas = pl.pallas_call(
        _fused_linears_kernel,
        grid=grid,
        in_specs=[
            pl.BlockSpec((_BN, d), lambda i: (i, 0)),
            pl.BlockSpec((d, kc), lambda i: (0, 0)),
            pl.BlockSpec((kc,), lambda i: (0,)),
            pl.BlockSpec((d, kb), lambda i: (0, 0)),
            pl.BlockSpec((kb,), lambda i: (0,)),
        ],
        out_specs=[
            pl.BlockSpec((_BN, kc), lambda i: (i, 0)),
            pl.BlockSpec((_BN, kb), lambda i: (i, 0)),
        ],
        out_shape=[
            jax.ShapeDtypeStruct((n, kc), jnp.float32),
            jax.ShapeDtypeStruct((n, kb), jnp.float32),
        ],
        compiler_params=pltpu.CompilerParams(
            dimension_semantics=("parallel",),
        ),
    )(x, W_cls, b_cls, W_box, b_box)
    return (scores, deltas)


# speedup vs baseline: 1.0015x; 1.0015x over previous
import jax
import jax.numpy as jnp
from jax.experimental import pallas as pl
from jax.experimental.pallas import tpu as pltpu

_BN = 1000
_NCHUNK = 4


def _fused_linears_kernel(*refs):
    x_refs = refs[:_NCHUNK]
    wc_ref, bc_ref, wb_ref, bb_ref, scores_ref, deltas_ref = refs[_NCHUNK:]
    dc = wc_ref.shape[0] // _NCHUNK
    acc_s = bc_ref[...] * jnp.ones((x_refs[0].shape[0], 1), jnp.float32)
    acc_d = bb_ref[...] * jnp.ones((x_refs[0].shape[0], 1), jnp.float32)
    for j in range(_NCHUNK):
        xj = x_refs[j][...].astype(jnp.bfloat16)
        wc_j = wc_ref[j * dc:(j + 1) * dc, :].astype(jnp.bfloat16)
        wb_j = wb_ref[j * dc:(j + 1) * dc, :].astype(jnp.bfloat16)
        acc_s = acc_s + jnp.dot(xj, wc_j, preferred_element_type=jnp.float32)
        acc_d = acc_d + jnp.dot(xj, wb_j, preferred_element_type=jnp.float32)
    scores_ref[...] = acc_s
    deltas_ref[...] = acc_d


@jax.jit
def kernel(x, W_cls, b_cls, W_box, b_box):
    if x.ndim > 2:
        x = x.reshape((x.shape[0], -1))
    n, d = x.shape
    kc = W_cls.shape[1]
    kb = W_box.shape[1]
    dc = d // _NCHUNK
    grid = (n // _BN,)
    x_specs = [
        pl.BlockSpec((_BN, dc), lambda i, j=j: (i, j)) for j in range(_NCHUNK)
    ]
    scores, deltas = pl.pallas_call(
        _fused_linears_kernel,
        grid=grid,
        in_specs=x_specs + [
            pl.BlockSpec((d, kc), lambda i: (0, 0)),
            pl.BlockSpec((kc,), lambda i: (0,)),
            pl.BlockSpec((d, kb), lambda i: (0, 0)),
            pl.BlockSpec((kb,), lambda i: (0,)),
        ],
        out_specs=[
            pl.BlockSpec((_BN, kc), lambda i: (i, 0)),
            pl.BlockSpec((_BN, kb), lambda i: (i, 0)),
        ],
        out_shape=[
            jax.ShapeDtypeStruct((n, kc), jnp.float32),
            jax.ShapeDtypeStruct((n, kb), jnp.float32),
        ],
        compiler_params=pltpu.CompilerParams(
            dimension_semantics=("arbitrary",),
        ),
    )(*([x] * _NCHUNK), W_cls, b_cls, W_box, b_box)
    return (scores, deltas)
